# Initial kernel scaffold; baseline (speedup 1.0000x reference)
import jax
import jax.numpy as jnp
from jax.experimental import pallas as pl


def kernel(x, pe_weight):
    raise NotImplementedError("write your pallas kernel here")


# speedup vs baseline: 5.0403x; 5.0403x over previous
import jax
import jax.numpy as jnp
from jax.experimental import pallas as pl

_ROWS = 512


def _bcast_body(w_ref, o_ref):
    o_ref[...] = jnp.broadcast_to(w_ref[...][None, :, :], o_ref.shape)


def kernel(x, pe_weight):
    B, L = x.shape
    M, D = pe_weight.shape
    return pl.pallas_call(
        _bcast_body,
        grid=(L // _ROWS,),
        in_specs=[pl.BlockSpec((_ROWS, D), lambda i: (i, 0))],
        out_specs=pl.BlockSpec((B, _ROWS, D), lambda i: (0, i, 0)),
        out_shape=jax.ShapeDtypeStruct((B, L, D), pe_weight.dtype),
    )(pe_weight)


# speedup vs baseline: 5.1937x; 1.0304x over previous
import jax
import jax.numpy as jnp
from jax.experimental import pallas as pl

_ROWS = 1024


def _bcast_body(w_ref, o_ref):
    o_ref[...] = jnp.broadcast_to(w_ref[...][None, :, :], o_ref.shape)


def kernel(x, pe_weight):
    B, L = x.shape
    M, D = pe_weight.shape
    return pl.pallas_call(
        _bcast_body,
        grid=(L // _ROWS,),
        in_specs=[pl.BlockSpec((_ROWS, D), lambda i: (i, 0))],
        out_specs=pl.BlockSpec((B, _ROWS, D), lambda i: (0, i, 0)),
        out_shape=jax.ShapeDtypeStruct((B, L, D), pe_weight.dtype),
    )(pe_weight)
